# baseline (speedup 1.0000x reference)
import jax
import jax.numpy as jnp
from jax import lax
from jax.experimental import pallas as pl
from jax.experimental.pallas import tpu as pltpu


N, D, C = 1024, 64, 32
CH = 256
NC = N // CH


def _fingerprint_kernel(
    adj_hbm, feat_ref, w_ref, out_ref, b0, b1, t0, t1, sem0, sem1
):
    bufs = (b0, b1)
    tb = (t0, t1)
    sems = (sem0, sem1)

    def dma(c):
        return pltpu.make_async_copy(
            adj_hbm.at[pl.ds(c * CH, CH), :], bufs[c % 2], sems[c % 2]
        )

    dma(0).start()
    dma(1).start()

    fw = jnp.dot(feat_ref[...], w_ref[...], preferred_element_type=jnp.float32)
    fwh = (0.5 * fw).astype(jnp.bfloat16)
    bias = 0.5 * jnp.sum(fw, axis=0, keepdims=True)
    eye = lax.broadcasted_iota(jnp.int32, (CH, CH), 0) == lax.broadcasted_iota(
        jnp.int32, (CH, CH), 1
    )

    def tanh_stage(c):
        dma(c).wait()
        buf = bufs[c % 2]
        buf[:, c * CH:(c + 1) * CH] = jnp.where(
            eye, -2e9, buf[:, c * CH:(c + 1) * CH]
        )
        tb[c % 2][...] = jnp.tanh((0.5 * buf[...]).astype(jnp.bfloat16))

    def matmul_stage(c):
        out_ref[c * CH:(c + 1) * CH, :] = (
            jnp.dot(tb[c % 2][...], fwh, preferred_element_type=jnp.float32) + bias
        )

    tanh_stage(0)
    for c in range(1, NC):
        tanh_stage(c)
        matmul_stage(c - 1)
        if c + 1 < NC:
            dma(c + 1).start()
    matmul_stage(NC - 1)


@jax.jit
def _run(adj_param, feat, W):
    return pl.pallas_call(
        _fingerprint_kernel,
        in_specs=[
            pl.BlockSpec(memory_space=pl.ANY),
            pl.BlockSpec(memory_space=pltpu.MemorySpace.VMEM),
            pl.BlockSpec(memory_space=pltpu.MemorySpace.VMEM),
        ],
        out_specs=pl.BlockSpec(memory_space=pltpu.MemorySpace.VMEM),
        out_shape=jax.ShapeDtypeStruct((N, C), jnp.float32),
        scratch_shapes=[
            pltpu.VMEM((CH, N), jnp.float32),
            pltpu.VMEM((CH, N), jnp.float32),
            pltpu.VMEM((CH, N), jnp.bfloat16),
            pltpu.VMEM((CH, N), jnp.bfloat16),
            pltpu.SemaphoreType.DMA,
            pltpu.SemaphoreType.DMA,
        ],
    )(adj_param, feat, W)


def kernel(feat, adj_param, edge_index_all, W):
    return _run(adj_param, feat, W)
